# baseline (speedup 1.0000x reference)
import jax
import jax.numpy as jnp
from jax import lax
from jax.experimental import pallas as pl
from jax.experimental.pallas import tpu as pltpu
from jax.experimental.pallas import tpu_sc as plsc

NUM_COLL = 4
EMBED_D = 64
BATCH = 16384
HIST = 50
LANES = 16
NCORE = 2
NSUB = 16
NWORK = NCORE * NSUB
RPW = BATCH // NWORK
FLAT = RPW * HIST
NBUF = 20
CHIDX = 10
SLOT = 16
NCH = FLAT // CHIDX
DCH = EMBED_D // LANES
INV_H = 1.0 / HIST


def _body(wq_hbm, idx_hbm, wr_hbm, out_hbm, idx_v, qbuf, rbuf, wr_v, out_v,
          *sems):
    wid = lax.axis_index("s") * NCORE + lax.axis_index("c")

    fbase = pl.multiple_of(wid * FLAT, 8)
    pltpu.sync_copy(idx_hbm.at[pl.ds(fbase, FLAT)], idx_v.at[pl.ds(0, FLAT)])
    pltpu.sync_copy(wr_hbm, wr_v)

    wrreg = [[wr_v[r, pl.ds(LANES * k, LANES)] for k in range(DCH)]
             for r in range(NUM_COLL)]
    lane = lax.iota(jnp.int32, LANES)
    tailmask = lane >= (3 * LANES - (HIST - LANES))
    offs = (0, LANES, 2 * LANES, HIST - LANES)
    def prep_chunk(ch, s):
        b = ch * CHIDX
        q = idx_v[pl.ds(b, LANES)] >> 2
        qbuf[pl.ds(s * SLOT, LANES)] = q

    def fire(ch, s):
        del ch
        pltpu.async_copy(wq_hbm.at[qbuf.at[pl.ds(s * SLOT, CHIDX)]],
                         rbuf.at[s], sems[s])

    def wait(s):
        pltpu.make_async_copy(wq_hbm.at[qbuf.at[pl.ds(s * SLOT, CHIDX)]],
                              rbuf.at[s], sems[s]).wait()

    def consume_row(row, slots):
        acc = [jnp.zeros((LANES,), jnp.float32) for _ in range(DCH)]
        for s in slots:
            for j in range(CHIDX):
                for k in range(DCH):
                    acc[k] = acc[k] + rbuf[s, j, pl.ds(LANES * k, LANES)]
        b = row * HIST
        rs = [idx_v[pl.ds(b + o, LANES)] & (NUM_COLL - 1) for o in offs]
        enc = jnp.zeros((LANES,), jnp.int32)
        for i in range(3):
            enc = enc + (jnp.int32(1) << (rs[i] << 3))
        enc = enc + jnp.where(tailmask, jnp.int32(1) << (rs[3] << 3), 0)
        for sh in (8, 4, 2, 1):
            enc = enc + enc.at[lane ^ sh].get(mode="promise_in_bounds",
                                              unique_indices=True)
        for r in range(NUM_COLL):
            cr = ((enc >> (8 * r)) & 255).astype(jnp.float32)
            for k in range(DCH):
                acc[k] = acc[k] + cr * wrreg[r][k]
        for k in range(DCH):
            out_v[row, pl.ds(LANES * k, LANES)] = acc[k] * INV_H

    for s in range(NBUF):
        prep_chunk(s, s)
        fire(s, s)

    CPR = HIST // CHIDX
    def outer(i, carry):
        base = i * NBUF
        for p in range(NBUF // CPR):
            slots = tuple(CPR * p + t for t in range(CPR))
            for s in slots:
                wait(s)
            consume_row(i * (NBUF // CPR) + p, slots)
            for s in slots:
                nxt = base + s + NBUF

                @pl.when(nxt < NCH)
                def _(s=s, nxt=nxt):
                    prep_chunk(nxt, s)
                    fire(nxt, s)
        return carry

    lax.fori_loop(0, NCH // NBUF, outer, 0)

    obase = pl.multiple_of(wid * RPW, 8)
    pltpu.sync_copy(out_v, out_hbm.at[pl.ds(obase, RPW)])


_mesh = plsc.VectorSubcoreMesh(core_axis_name="c", subcore_axis_name="s",
                               num_cores=NCORE, num_subcores=NSUB)

_sc_call = pl.kernel(
    _body,
    out_type=jax.ShapeDtypeStruct((BATCH, EMBED_D), jnp.float32),
    mesh=_mesh,
    compiler_params=pltpu.CompilerParams(use_tc_tiling_on_sc=False),
    scratch_types=[
        pltpu.VMEM((FLAT + LANES,), jnp.int32),
        pltpu.VMEM((NBUF * SLOT,), jnp.int32),
        pltpu.VMEM((NBUF, CHIDX, EMBED_D), jnp.float32),
        pltpu.VMEM((NUM_COLL, EMBED_D), jnp.float32),
        pltpu.VMEM((RPW, EMBED_D), jnp.float32),
    ] + [pltpu.SemaphoreType.DMA] * NBUF,
)


@jax.jit
def kernel(input, weight_q, weight_r):
    idx_flat = input.astype(jnp.int32).reshape(-1)
    return _sc_call(weight_q, idx_flat, weight_r)


# speedup vs baseline: 1.3442x; 1.3442x over previous
import jax
import jax.numpy as jnp
from jax import lax
from jax.experimental import pallas as pl
from jax.experimental.pallas import tpu as pltpu
from jax.experimental.pallas import tpu_sc as plsc

NUM_COLL = 4
EMBED_D = 64
BATCH = 16384
HIST = 50
LANES = 16
NCORE = 2
NSUB = 16
NWORK = NCORE * NSUB
RPW = BATCH // NWORK
FLAT = RPW * HIST
NBUF = 8
CHIDX = 25
SLOT = 32
NCH = FLAT // CHIDX
DCH = EMBED_D // LANES
INV_H = 1.0 / HIST


def _body(wq_hbm, idx_hbm, wr_hbm, out_hbm, idx_v, qbuf, rbuf, wr_v, out_v,
          *sems):
    wid = lax.axis_index("s") * NCORE + lax.axis_index("c")

    fbase = pl.multiple_of(wid * FLAT, 8)
    pltpu.sync_copy(idx_hbm.at[pl.ds(fbase, FLAT)], idx_v)
    pltpu.sync_copy(wr_hbm, wr_v)

    wrreg = [[wr_v[r, pl.ds(LANES * k, LANES)] for k in range(DCH)]
             for r in range(NUM_COLL)]
    lane = lax.iota(jnp.int32, LANES)
    tailmask = lane >= (3 * LANES - (HIST - LANES))
    offs = (0, LANES, 2 * LANES, HIST - LANES)
    ch_offs = (0, CHIDX - LANES)

    def prep_chunk(ch, s):
        b = ch * CHIDX
        for o in ch_offs:
            q = idx_v[pl.ds(b + o, LANES)] >> 2
            qbuf[pl.ds(s * SLOT + o, LANES)] = q

    def fire(ch, s):
        del ch
        pltpu.async_copy(wq_hbm.at[qbuf.at[pl.ds(s * SLOT, CHIDX)]],
                         rbuf.at[s], sems[s])

    def wait(s):
        pltpu.make_async_copy(wq_hbm.at[qbuf.at[pl.ds(s * SLOT, CHIDX)]],
                              rbuf.at[s], sems[s]).wait()

    def consume_row(row, s0, s1):
        acc = [jnp.zeros((LANES,), jnp.float32) for _ in range(DCH)]
        for s in (s0, s1):
            for j in range(CHIDX):
                for k in range(DCH):
                    acc[k] = acc[k] + rbuf[s, j, pl.ds(LANES * k, LANES)]
        b = row * HIST
        rs = [idx_v[pl.ds(b + o, LANES)] & (NUM_COLL - 1) for o in offs]
        enc = jnp.zeros((LANES,), jnp.int32)
        for i in range(3):
            enc = enc + (jnp.int32(1) << (rs[i] << 3))
        enc = enc + jnp.where(tailmask, jnp.int32(1) << (rs[3] << 3), 0)
        for sh in (8, 4, 2, 1):
            enc = enc + enc.at[lane ^ sh].get(mode="promise_in_bounds",
                                              unique_indices=True)
        for r in range(NUM_COLL):
            cr = ((enc >> (8 * r)) & 255).astype(jnp.float32)
            for k in range(DCH):
                acc[k] = acc[k] + cr * wrreg[r][k]
        for k in range(DCH):
            out_v[row, pl.ds(LANES * k, LANES)] = acc[k] * INV_H

    for s in range(NBUF):
        prep_chunk(s, s)
        fire(s, s)

    def outer(i, carry):
        base = i * NBUF
        for p in range(NBUF // 2):
            s0, s1 = 2 * p, 2 * p + 1
            wait(s0)
            wait(s1)
            consume_row(i * (NBUF // 2) + p, s0, s1)
            for s, nxt in ((s0, base + s0 + NBUF), (s1, base + s1 + NBUF)):
                @pl.when(nxt < NCH)
                def _(s=s, nxt=nxt):
                    prep_chunk(nxt, s)
                    fire(nxt, s)
        return carry

    lax.fori_loop(0, NCH // NBUF, outer, 0)

    obase = pl.multiple_of(wid * RPW, 8)
    pltpu.sync_copy(out_v, out_hbm.at[pl.ds(obase, RPW)])


_mesh = plsc.VectorSubcoreMesh(core_axis_name="c", subcore_axis_name="s",
                               num_cores=NCORE, num_subcores=NSUB)

_sc_call = pl.kernel(
    _body,
    out_type=jax.ShapeDtypeStruct((BATCH, EMBED_D), jnp.float32),
    mesh=_mesh,
    compiler_params=pltpu.CompilerParams(use_tc_tiling_on_sc=False),
    scratch_types=[
        pltpu.VMEM((FLAT,), jnp.int32),
        pltpu.VMEM((NBUF * SLOT,), jnp.int32),
        pltpu.VMEM((NBUF, CHIDX, EMBED_D), jnp.float32),
        pltpu.VMEM((NUM_COLL, EMBED_D), jnp.float32),
        pltpu.VMEM((RPW, EMBED_D), jnp.float32),
    ] + [pltpu.SemaphoreType.DMA] * NBUF,
)


@jax.jit
def kernel(input, weight_q, weight_r):
    idx_flat = input.astype(jnp.int32).reshape(-1)
    return _sc_call(weight_q, idx_flat, weight_r)
